# baseline (speedup 1.0000x reference)
import jax
import jax.numpy as jnp
from jax.experimental import pallas as pl
from jax.experimental.pallas import tpu as pltpu

_N = 512
_O = 256
_BB = 8
_EPS = 3e-5


def _eq_kernel(s_ref, w_ref, tri_ref, out_ref):
    w = w_ref[...]
    tri = tri_ref[...]

    dot = lambda a, b: jnp.dot(a, b, preferred_element_type=jnp.float32)

    for i in range(_BB):
        s_row = s_ref[i]
        s_col = jnp.transpose(s_row)
        e1_row = jnp.exp(s_row)
        e1_col = jnp.transpose(e1_row)

        m = (s_row < s_col) | ((s_row == s_col) & tri)
        mb = jnp.where(m, 1.0, 0.0).astype(jnp.bfloat16)

        u1 = e1_col * w
        u2 = e1_col * u1
        u1h = u1.astype(jnp.bfloat16)
        u1l = (u1 - u1h.astype(jnp.float32)).astype(jnp.bfloat16)
        u2h = u2.astype(jnp.bfloat16)
        u2l = (u2 - u2h.astype(jnp.float32)).astype(jnp.bfloat16)

        rh = jnp.concatenate([u1h, u2h], axis=1)
        rl = jnp.concatenate([u1l, u2l], axis=1)
        a12 = dot(mb, rh) + dot(mb, rl)
        a1 = a12[:, :_O]
        a2 = a12[:, _O:]

        next_e1 = jnp.min(
            jnp.where(m, jnp.inf, jnp.broadcast_to(e1_row, (_N, _N))),
            axis=1, keepdims=True)

        disc = a1 * a1 - 4.0 * a2
        valid = disc > 0.0
        sqrt_d = jnp.sqrt(jnp.where(valid, disc, 1.0))
        ratio = 0.5 * (a1 - sqrt_d)

        lo = e1_col * (1.0 - _EPS)
        hi = next_e1 * (1.0 + _EPS)
        ok = valid & (ratio >= lo) & (ratio <= hi)
        cand = jnp.where(ok, ratio, jnp.inf)
        out_ref[i] = jnp.log(jnp.min(cand, axis=0, keepdims=True))


def kernel(input_spikes, input_weights):
    batch = input_spikes.shape[0]
    s3 = input_spikes.reshape(batch, 1, _N)
    idx = jnp.arange(_N)
    tri = idx[None, :] <= idx[:, None]
    out = pl.pallas_call(
        _eq_kernel,
        grid=(batch // _BB,),
        in_specs=[
            pl.BlockSpec((_BB, 1, _N), lambda b: (b, 0, 0)),
            pl.BlockSpec((_N, _O), lambda b: (0, 0)),
            pl.BlockSpec((_N, _N), lambda b: (0, 0)),
        ],
        out_specs=pl.BlockSpec((_BB, 1, _O), lambda b: (b, 0, 0)),
        out_shape=jax.ShapeDtypeStruct((batch, 1, _O), jnp.float32),
        compiler_params=pltpu.CompilerParams(
            dimension_semantics=("parallel",)),
    )(s3, input_weights, tri)
    return out.reshape(batch, _O)


# speedup vs baseline: 1.0547x; 1.0547x over previous
import jax
import jax.numpy as jnp
from jax.experimental import pallas as pl
from jax.experimental.pallas import tpu as pltpu

_N = 512
_O = 256
_BB = 16
_EPS = 3e-5


def _eq_kernel(s_ref, w_ref, tri_ref, out_ref):
    w = w_ref[...]
    tri = tri_ref[...]

    dot = lambda a, b: jnp.dot(a, b, preferred_element_type=jnp.float32)

    for i in range(_BB):
        s_row = s_ref[i]
        s_col = jnp.transpose(s_row)
        e1_row = jnp.exp(s_row)
        e1_col = jnp.transpose(e1_row)

        m = (s_row < s_col) | ((s_row == s_col) & tri)
        mb = jnp.where(m, 1.0, 0.0).astype(jnp.bfloat16)

        u1 = e1_col * w
        u2 = e1_col * u1
        u1h = u1.astype(jnp.bfloat16)
        u1l = (u1 - u1h.astype(jnp.float32)).astype(jnp.bfloat16)
        u2h = u2.astype(jnp.bfloat16)
        u2l = (u2 - u2h.astype(jnp.float32)).astype(jnp.bfloat16)

        rh = jnp.concatenate([u1h, u2h], axis=1)
        rl = jnp.concatenate([u1l, u2l], axis=1)
        a12 = dot(mb, rh) + dot(mb, rl)
        a1 = a12[:, :_O]
        a2 = a12[:, _O:]

        next_e1 = jnp.min(
            jnp.where(m, jnp.inf, jnp.broadcast_to(e1_row, (_N, _N))),
            axis=1, keepdims=True)

        disc = a1 * a1 - 4.0 * a2
        sqrt_d = jnp.sqrt(jnp.where(disc > 0.0, disc, 1e30))
        ratio = 0.5 * (a1 - sqrt_d)

        lo = e1_col * (1.0 - _EPS)
        hi = next_e1 * (1.0 + _EPS)
        ok = (ratio >= lo) & (ratio <= hi)
        cand = jnp.where(ok, ratio, jnp.inf)
        out_ref[i] = jnp.log(jnp.min(cand, axis=0, keepdims=True))


def kernel(input_spikes, input_weights):
    batch = input_spikes.shape[0]
    s3 = input_spikes.reshape(batch, 1, _N)
    idx = jnp.arange(_N)
    tri = idx[None, :] <= idx[:, None]
    out = pl.pallas_call(
        _eq_kernel,
        grid=(batch // _BB,),
        in_specs=[
            pl.BlockSpec((_BB, 1, _N), lambda b: (b, 0, 0)),
            pl.BlockSpec((_N, _O), lambda b: (0, 0)),
            pl.BlockSpec((_N, _N), lambda b: (0, 0)),
        ],
        out_specs=pl.BlockSpec((_BB, 1, _O), lambda b: (b, 0, 0)),
        out_shape=jax.ShapeDtypeStruct((batch, 1, _O), jnp.float32),
        compiler_params=pltpu.CompilerParams(
            dimension_semantics=("parallel",)),
    )(s3, input_weights, tri)
    return out.reshape(batch, _O)


# speedup vs baseline: 1.0773x; 1.0214x over previous
import jax
import jax.numpy as jnp
from jax.experimental import pallas as pl
from jax.experimental.pallas import tpu as pltpu

_N = 512
_O = 256
_BB = 16
_EPS = 3e-5


def _eq_kernel(s_ref, w_ref, tri_ref, out_ref):
    w = w_ref[...]
    tri = tri_ref[...]

    dot = lambda a, b: jnp.dot(a, b, preferred_element_type=jnp.float32)

    for i in range(_BB):
        s_row = s_ref[i]
        s_col = jnp.transpose(s_row)
        e1_row = jnp.exp(s_row)
        e1_col = jnp.transpose(e1_row)

        m = (s_row < s_col) | ((s_row == s_col) & tri)
        mb = jnp.where(m, 1.0, 0.0).astype(jnp.bfloat16)

        u1 = e1_col * w
        u2 = e1_col * u1
        u1h = u1.astype(jnp.bfloat16)
        u1l = (u1 - u1h.astype(jnp.float32)).astype(jnp.bfloat16)
        u2h = u2.astype(jnp.bfloat16)
        u2l = (u2 - u2h.astype(jnp.float32)).astype(jnp.bfloat16)

        rh = jnp.concatenate([u1h, u2h], axis=1)
        rl = jnp.concatenate([u1l, u2l], axis=1)
        a12 = dot(mb, rh) + dot(mb, rl)
        a1 = a12[:, :_O]
        a2 = a12[:, _O:]

        next_e1 = jnp.min(
            jnp.where(m, jnp.inf, jnp.broadcast_to(e1_row, (_N, _N))),
            axis=1, keepdims=True)

        disc = a1 * a1 - 4.0 * a2
        ratio = 0.5 * (a1 - jnp.sqrt(disc))

        lo = e1_col * (1.0 - _EPS)
        hi = next_e1 * (1.0 + _EPS)
        ok = (ratio >= lo) & (ratio <= hi)
        cand = jnp.where(ok, ratio, jnp.inf)
        out_ref[i] = jnp.log(jnp.min(cand, axis=0, keepdims=True))


def kernel(input_spikes, input_weights):
    batch = input_spikes.shape[0]
    s3 = input_spikes.reshape(batch, 1, _N)
    idx = jnp.arange(_N)
    tri = idx[None, :] <= idx[:, None]
    out = pl.pallas_call(
        _eq_kernel,
        grid=(batch // _BB,),
        in_specs=[
            pl.BlockSpec((_BB, 1, _N), lambda b: (b, 0, 0)),
            pl.BlockSpec((_N, _O), lambda b: (0, 0)),
            pl.BlockSpec((_N, _N), lambda b: (0, 0)),
        ],
        out_specs=pl.BlockSpec((_BB, 1, _O), lambda b: (b, 0, 0)),
        out_shape=jax.ShapeDtypeStruct((batch, 1, _O), jnp.float32),
        compiler_params=pltpu.CompilerParams(
            dimension_semantics=("parallel",)),
    )(s3, input_weights, tri)
    return out.reshape(batch, _O)


# speedup vs baseline: 1.2934x; 1.2006x over previous
import jax
import jax.numpy as jnp
from jax.experimental import pallas as pl
from jax.experimental.pallas import tpu as pltpu

_N = 512
_O = 256
_BB = 16
_T = 128
_EPS = 3e-5


def _eq_kernel(s_ref, w_ref, tri_ref, out_ref):
    w = w_ref[...]
    tri = tri_ref[...]

    dot = lambda a, b: jnp.dot(a, b, preferred_element_type=jnp.float32)

    for i in range(_BB):
        s_row = s_ref[i]
        s_col = jnp.transpose(s_row)
        e1_row = jnp.exp(s_row)
        e1_col = jnp.transpose(e1_row)

        row_tiles = []
        for t in range(_N // _T):
            sk_t = s_col[t * _T:(t + 1) * _T]
            parts = []
            for u in range(_N // _T):
                sj_u = s_row[:, u * _T:(u + 1) * _T]
                if u < t:
                    parts.append(sj_u <= sk_t)
                elif u > t:
                    parts.append(sj_u < sk_t)
                else:
                    parts.append((sj_u < sk_t) | ((sj_u == sk_t) & tri))
            row_tiles.append(jnp.concatenate(parts, axis=1))
        m = jnp.concatenate(row_tiles, axis=0)
        mb = jnp.where(m, 1.0, 0.0).astype(jnp.bfloat16)

        u1 = e1_col * w
        u2 = e1_col * u1
        u1h = u1.astype(jnp.bfloat16)
        u1l = (u1 - u1h.astype(jnp.float32)).astype(jnp.bfloat16)
        u2h = u2.astype(jnp.bfloat16)
        u2l = (u2 - u2h.astype(jnp.float32)).astype(jnp.bfloat16)

        rh = jnp.concatenate([u1h, u2h], axis=1)
        rl = jnp.concatenate([u1l, u2l], axis=1)
        a12 = dot(mb, rh) + dot(mb, rl)
        a1 = a12[:, :_O]
        a2 = a12[:, _O:]

        next_e1 = jnp.min(
            jnp.where(m, jnp.inf, jnp.broadcast_to(e1_row, (_N, _N))),
            axis=1, keepdims=True)

        disc = a1 * a1 - 4.0 * a2
        ratio = 0.5 * (a1 - jnp.sqrt(disc))

        lo = e1_col * (1.0 - _EPS)
        hi = next_e1 * (1.0 + _EPS)
        ok = (ratio >= lo) & (ratio <= hi)
        cand = jnp.where(ok, ratio, jnp.inf)
        out_ref[i] = jnp.log(jnp.min(cand, axis=0, keepdims=True))


def kernel(input_spikes, input_weights):
    batch = input_spikes.shape[0]
    s3 = input_spikes.reshape(batch, 1, _N)
    idx = jnp.arange(_T)
    tri = idx[None, :] <= idx[:, None]
    out = pl.pallas_call(
        _eq_kernel,
        grid=(batch // _BB,),
        in_specs=[
            pl.BlockSpec((_BB, 1, _N), lambda b: (b, 0, 0)),
            pl.BlockSpec((_N, _O), lambda b: (0, 0)),
            pl.BlockSpec((_T, _T), lambda b: (0, 0)),
        ],
        out_specs=pl.BlockSpec((_BB, 1, _O), lambda b: (b, 0, 0)),
        out_shape=jax.ShapeDtypeStruct((batch, 1, _O), jnp.float32),
        compiler_params=pltpu.CompilerParams(
            dimension_semantics=("parallel",)),
    )(s3, input_weights, tri)
    return out.reshape(batch, _O)
